# baseline (speedup 1.0000x reference)
import functools

import jax
import jax.numpy as jnp
from jax.experimental import pallas as pl
from jax.experimental.pallas import tpu as pltpu

E = 64
TOPK = 2
H = 768
I = 768
S = 2048
N = S * TOPK
LIMIT = 7.0
ALPHA = 1.702

BT = 128
G = N // BT + E


def _moe_body(tile_e_ref, tot_ref, tokens_ref,
              w_ref, x_ref, bgu_ref, bdn_ref, wgu_hbm, wdn_hbm,
              out_ref,
              wgu_buf, wdn_buf, xs_ref, ys_ref, dsem):
    T = tot_ref[0]
    out_ref[...] = jnp.zeros_like(out_ref)

    CGU, CDN = 4, 2
    RGU, RDN = H // CGU, I // CDN

    def _copies(i, slot):
        e = tile_e_ref[i]
        cps = []
        for c in range(CGU):
            cps.append(pltpu.make_async_copy(
                wgu_hbm.at[e, pl.ds(c * RGU, RGU)],
                wgu_buf.at[slot, pl.ds(c * RGU, RGU)],
                dsem.at[slot, c]))
        for c in range(CDN):
            cps.append(pltpu.make_async_copy(
                wdn_hbm.at[e, pl.ds(c * RDN, RDN)],
                wdn_buf.at[slot, pl.ds(c * RDN, RDN)],
                dsem.at[slot, CGU + c]))
        return cps

    def start_copy(i, slot):
        for cp in _copies(i, slot):
            cp.start()

    def wait_copy(i, slot):
        for cp in _copies(i, slot):
            cp.wait()

    start_copy(0, 0)

    def tile_body(i, carry):
        slot = jax.lax.rem(i, 2)

        @pl.when(i + 1 < T)
        def _prefetch():
            start_copy(i + 1, 1 - slot)

        def gather_row(r, c):
            t = tokens_ref[i, r]
            xs_ref[r, :] = x_ref[t, :]
            return c
        jax.lax.fori_loop(0, BT, gather_row, 0, unroll=8)

        wait_copy(i, slot)
        e = tile_e_ref[i]
        xs = xs_ref[...]
        wgu = wgu_buf[slot]
        gu = jnp.dot(xs, wgu, preferred_element_type=jnp.float32)
        gu = gu + bgu_ref[e, :][None, :]
        gate = jnp.minimum(gu[:, :I], LIMIT)
        up = jnp.clip(gu[:, I:], -LIMIT, LIMIT)
        glu = gate * jax.nn.sigmoid(gate * ALPHA)
        h = (up + 1.0) * glu
        y = jnp.dot(h, wdn_buf[slot], preferred_element_type=jnp.float32)
        y = y + bdn_ref[e, :][None, :]
        ys_ref[...] = y * w_ref[i, 0, :][:, None]

        def scatter_row(r, c):
            t = tokens_ref[i, r]
            out_ref[pl.ds(t, 1), :] += ys_ref[pl.ds(r, 1), :]
            return c
        jax.lax.fori_loop(0, BT, scatter_row, 0, unroll=8)
        return carry

    jax.lax.fori_loop(0, T, tile_body, 0)


@functools.partial(jax.jit, static_argnames=())
def kernel(hidden_states, router_indices, routing_weights,
           W_gate_up, b_gate_up, W_down, b_down):
    x = hidden_states[0]
    experts = router_indices.reshape(N).astype(jnp.int32)
    w_flat = routing_weights.reshape(N)

    order = jnp.argsort(experts, stable=True)
    tok_sorted = (order // TOPK).astype(jnp.int32)
    w_sorted = w_flat[order]
    counts = jnp.bincount(experts, length=E).astype(jnp.int32)
    offsets = jnp.concatenate([jnp.zeros((1,), jnp.int32),
                               jnp.cumsum(counts)[:-1].astype(jnp.int32)])
    nt = (counts + BT - 1) // BT
    cum_nt = jnp.cumsum(nt).astype(jnp.int32)
    first_tile = cum_nt - nt
    total_tiles = cum_nt[-1:]
    gids = jnp.arange(G, dtype=jnp.int32)
    tile_e = jnp.searchsorted(cum_nt, gids, side='right').astype(jnp.int32)
    tile_e = jnp.minimum(tile_e, E - 1)
    tile_local = gids - first_tile[tile_e]
    tile_start = offsets[tile_e] + tile_local * BT
    tile_cnt = jnp.clip(counts[tile_e] - tile_local * BT, 0, BT).astype(jnp.int32)

    row_ids = tile_start[:, None] + jnp.arange(BT, dtype=jnp.int32)[None, :]
    row_valid = jnp.arange(BT, dtype=jnp.int32)[None, :] < tile_cnt[:, None]
    row_ids = jnp.clip(row_ids, 0, N - 1)
    tokens_tile = jnp.where(row_valid, tok_sorted[row_ids], 0)
    w_tile = jnp.where(row_valid, w_sorted[row_ids], 0.0)
    w_tile = w_tile.reshape(G, 1, BT)

    grid_spec = pltpu.PrefetchScalarGridSpec(
        num_scalar_prefetch=3,
        grid=(1,),
        in_specs=[
            pl.BlockSpec((G, 1, BT), lambda g, te, tot, tok: (0, 0, 0)),
            pl.BlockSpec((S, H), lambda g, te, tot, tok: (0, 0)),
            pl.BlockSpec((E, 2 * I), lambda g, te, tot, tok: (0, 0)),
            pl.BlockSpec((E, H), lambda g, te, tot, tok: (0, 0)),
            pl.BlockSpec(memory_space=pl.ANY),
            pl.BlockSpec(memory_space=pl.ANY),
        ],
        out_specs=pl.BlockSpec((S, H), lambda g, te, tot, tok: (0, 0)),
        scratch_shapes=[
            pltpu.VMEM((2, H, 2 * I), jnp.float32),
            pltpu.VMEM((2, I, H), jnp.float32),
            pltpu.VMEM((BT, H), jnp.float32),
            pltpu.VMEM((BT, H), jnp.float32),
            pltpu.SemaphoreType.DMA((2, 6)),
        ],
    )

    out = pl.pallas_call(
        _moe_body,
        grid_spec=grid_spec,
        out_shape=jax.ShapeDtypeStruct((S, H), jnp.float32),
        compiler_params=pltpu.CompilerParams(
            dimension_semantics=("arbitrary",),
        ),
    )(tile_e, total_tiles, tokens_tile,
      w_tile, x, b_gate_up, b_down, W_gate_up, W_down)

    return out.reshape(1, S, H)


# speedup vs baseline: 1.6613x; 1.6613x over previous
import functools

import jax
import jax.numpy as jnp
from jax.experimental import pallas as pl
from jax.experimental.pallas import tpu as pltpu

E = 64
TOPK = 2
H = 768
I = 768
S = 2048
N = S * TOPK
LIMIT = 7.0
ALPHA = 1.702

BT = 128
G = N // BT + E
NR = N // 128


def _meta_body(ex_ref, dest_ref, te_ref, tc_ref, tot_ref):
    ex = ex_ref[...]
    dest = jnp.zeros((NR, 128), jnp.int32)
    te = jnp.zeros((1, G), jnp.int32)
    tc = jnp.zeros((1, G), jnp.int32)
    lanes_g = jax.lax.broadcasted_iota(jnp.int32, (1, G), 1)
    ft = jnp.zeros((1, 1), jnp.int32)

    for e in range(E):
        m = (ex == e).astype(jnp.int32)
        c = m
        for k in (1, 2, 4, 8, 16, 32, 64):
            c = c + jnp.concatenate(
                [jnp.zeros((NR, k), jnp.int32), c[:, :128 - k]], axis=1)
        rs = c[:, 127:128]
        rc = rs
        kk = 1
        while kk < NR:
            rc = rc + jnp.concatenate(
                [jnp.zeros((kk, 1), jnp.int32), rc[:NR - kk]], axis=0)
            kk *= 2
        rank = c - m + (rc - rs)
        cnt = rc[NR - 1:NR, 0:1]
        nt = (cnt + BT - 1) >> 7
        dest = dest + m * (ft * BT + rank)
        in_e = (lanes_g >= ft) & (lanes_g < ft + nt)
        te = jnp.where(in_e, e, te)
        tc = jnp.where(in_e,
                       jnp.clip(cnt - (lanes_g - ft) * BT, 0, BT), tc)
        ft = ft + nt

    dest_ref[...] = dest
    te_ref[...] = te
    tc_ref[...] = tc
    tot_ref[...] = ft


def _moe_body(te_ref, tc_ref, tot_ref, dest_ref, rw_ref,
              x_ref, bgu_ref, bdn_ref, wgu_hbm, wdn_hbm,
              out_ref,
              wgu_buf, wdn_buf, xs_ref, ys_ref,
              tok_sm, ws_sm, dsem):
    T = tot_ref[0, 0]
    out_ref[...] = jnp.zeros_like(out_ref)

    CGU, CDN = 4, 2
    RGU, RDN = H // CGU, I // CDN

    def _copies(i, slot):
        e = te_ref[0, i]
        cps = []
        for c in range(CGU):
            cps.append(pltpu.make_async_copy(
                wgu_hbm.at[e, pl.ds(c * RGU, RGU)],
                wgu_buf.at[slot, pl.ds(c * RGU, RGU)],
                dsem.at[slot, c]))
        for c in range(CDN):
            cps.append(pltpu.make_async_copy(
                wdn_hbm.at[e, pl.ds(c * RDN, RDN)],
                wdn_buf.at[slot, pl.ds(c * RDN, RDN)],
                dsem.at[slot, CGU + c]))
        return cps

    def start_copy(i, slot):
        for cp in _copies(i, slot):
            cp.start()

    def wait_copy(i, slot):
        for cp in _copies(i, slot):
            cp.wait()

    start_copy(0, 0)

    def invert(p, c):
        hi = jax.lax.shift_right_logical(p, 7)
        lo = jax.lax.bitwise_and(p, 127)
        s = dest_ref[hi, lo]
        tok_sm[s] = jax.lax.shift_right_logical(p, 1)
        ws_sm[s] = rw_ref[hi, lo]
        return c
    jax.lax.fori_loop(0, N, invert, 0, unroll=4)

    def tile_body(i, carry):
        slot = jax.lax.rem(i, 2)
        cnt = tc_ref[0, i]
        base = i * BT

        @pl.when(i + 1 < T)
        def _prefetch():
            start_copy(i + 1, 1 - slot)

        def gather_row(r, c):
            t = tok_sm[base + r]
            xs_ref[pl.ds(r, 1), :] = x_ref[pl.ds(t, 1), :]
            return c
        jax.lax.fori_loop(0, cnt, gather_row, 0)

        wait_copy(i, slot)
        e = te_ref[0, i]
        xs = xs_ref[...]
        gu = jnp.dot(xs, wgu_buf[slot], preferred_element_type=jnp.float32)
        gu = gu + bgu_ref[e, :][None, :]
        gate = jnp.minimum(gu[:, :I], LIMIT)
        up = jnp.clip(gu[:, I:], -LIMIT, LIMIT)
        glu = gate * jax.nn.sigmoid(gate * ALPHA)
        h = (up + 1.0) * glu
        y = jnp.dot(h, wdn_buf[slot], preferred_element_type=jnp.float32)
        y = y + bdn_ref[e, :][None, :]
        ys_ref[...] = y

        def scatter_row(r, c):
            t = tok_sm[base + r]
            w = ws_sm[base + r]
            out_ref[pl.ds(t, 1), :] += ys_ref[pl.ds(r, 1), :] * w
            return c
        jax.lax.fori_loop(0, cnt, scatter_row, 0)
        return carry

    jax.lax.fori_loop(0, T, tile_body, 0)


@functools.partial(jax.jit, static_argnames=())
def kernel(hidden_states, router_indices, routing_weights,
           W_gate_up, b_gate_up, W_down, b_down):
    x = hidden_states[0]
    ex2d = router_indices.astype(jnp.int32).reshape(NR, 128)

    dest, te, tc, tot = pl.pallas_call(
        _meta_body,
        out_shape=[
            jax.ShapeDtypeStruct((NR, 128), jnp.int32),
            jax.ShapeDtypeStruct((1, G), jnp.int32),
            jax.ShapeDtypeStruct((1, G), jnp.int32),
            jax.ShapeDtypeStruct((1, 1), jnp.int32),
        ],
    )(ex2d)

    grid_spec = pltpu.PrefetchScalarGridSpec(
        num_scalar_prefetch=5,
        grid=(1,),
        in_specs=[
            pl.BlockSpec((S, H), lambda g, *s: (0, 0)),
            pl.BlockSpec((E, 2 * I), lambda g, *s: (0, 0)),
            pl.BlockSpec((E, H), lambda g, *s: (0, 0)),
            pl.BlockSpec(memory_space=pl.ANY),
            pl.BlockSpec(memory_space=pl.ANY),
        ],
        out_specs=pl.BlockSpec((S, H), lambda g, *s: (0, 0)),
        scratch_shapes=[
            pltpu.VMEM((2, H, 2 * I), jnp.float32),
            pltpu.VMEM((2, I, H), jnp.float32),
            pltpu.VMEM((BT, H), jnp.float32),
            pltpu.VMEM((BT, H), jnp.float32),
            pltpu.SMEM((G * BT,), jnp.int32),
            pltpu.SMEM((G * BT,), jnp.float32),
            pltpu.SemaphoreType.DMA((2, 6)),
        ],
    )

    out = pl.pallas_call(
        _moe_body,
        grid_spec=grid_spec,
        out_shape=jax.ShapeDtypeStruct((S, H), jnp.float32),
        compiler_params=pltpu.CompilerParams(
            dimension_semantics=("arbitrary",),
        ),
    )(te, tc, tot, dest, routing_weights.reshape(NR, 128),
      x, b_gate_up, b_down, W_gate_up, W_down)

    return out.reshape(1, S, H)


# speedup vs baseline: 1.7436x; 1.0496x over previous
import functools

import jax
import jax.numpy as jnp
from jax.experimental import pallas as pl
from jax.experimental.pallas import tpu as pltpu

E = 64
TOPK = 2
H = 768
I = 768
S = 2048
N = S * TOPK
LIMIT = 7.0
ALPHA = 1.702

BT = 128
G = N // BT + E
NR = N // 128


def _meta_body(ex_ref, dest_ref, te_ref, tc_ref, tot_ref):
    ex = ex_ref[...]
    dest = jnp.zeros((NR, 128), jnp.int32)
    te = jnp.zeros((1, G), jnp.int32)
    tc = jnp.zeros((1, G), jnp.int32)
    lanes_g = jax.lax.broadcasted_iota(jnp.int32, (1, G), 1)
    ft = jnp.zeros((1, 1), jnp.int32)

    for e in range(E):
        m = (ex == e).astype(jnp.int32)
        c = m
        for k in (1, 2, 4, 8, 16, 32, 64):
            c = c + jnp.concatenate(
                [jnp.zeros((NR, k), jnp.int32), c[:, :128 - k]], axis=1)
        rs = c[:, 127:128]
        rc = rs
        kk = 1
        while kk < NR:
            rc = rc + jnp.concatenate(
                [jnp.zeros((kk, 1), jnp.int32), rc[:NR - kk]], axis=0)
            kk *= 2
        rank = c - m + (rc - rs)
        cnt = rc[NR - 1:NR, 0:1]
        nt = (cnt + BT - 1) >> 7
        dest = dest + m * (ft * BT + rank)
        in_e = (lanes_g >= ft) & (lanes_g < ft + nt)
        te = jnp.where(in_e, e, te)
        tc = jnp.where(in_e,
                       jnp.clip(cnt - (lanes_g - ft) * BT, 0, BT), tc)
        ft = ft + nt

    dest_ref[...] = dest
    te_ref[...] = te
    tc_ref[...] = tc
    tot_ref[...] = ft


def _moe_body(te_ref, tc_ref, tot_ref, dest_ref, rw_ref,
              x_ref, bgu_ref, bdn_ref, wgu_hbm, wdn_hbm,
              out_ref,
              wgu_buf, wdn_buf, xs_ref, ys_ref,
              tok_sm, ws_sm, dsem):
    T = tot_ref[0, 0]
    out_ref[...] = jnp.zeros_like(out_ref)

    CGU, CDN = 4, 2
    RGU, RDN = H // CGU, I // CDN

    def _copies(i, slot):
        e = te_ref[0, i]
        cps = []
        for c in range(CGU):
            cps.append(pltpu.make_async_copy(
                wgu_hbm.at[e, pl.ds(c * RGU, RGU)],
                wgu_buf.at[slot, pl.ds(c * RGU, RGU)],
                dsem.at[slot, c]))
        for c in range(CDN):
            cps.append(pltpu.make_async_copy(
                wdn_hbm.at[e, pl.ds(c * RDN, RDN)],
                wdn_buf.at[slot, pl.ds(c * RDN, RDN)],
                dsem.at[slot, CGU + c]))
        return cps

    def start_copy(i, slot):
        for cp in _copies(i, slot):
            cp.start()

    def wait_copy(i, slot):
        for cp in _copies(i, slot):
            cp.wait()

    start_copy(0, 0)
    for j in (1, 2):
        @pl.when(j < T)
        def _pro():
            start_copy(j, j)

    def invert(p, c):
        hi = jax.lax.shift_right_logical(p, 7)
        lo = jax.lax.bitwise_and(p, 127)
        s = dest_ref[hi, lo]
        tok_sm[s] = jax.lax.shift_right_logical(p, 1)
        ws_sm[s] = rw_ref[hi, lo]
        return c
    jax.lax.fori_loop(0, N, invert, 0, unroll=4)

    def tile_body(i, carry):
        slot = jax.lax.rem(i, 4)
        cnt = tc_ref[0, i]
        base = i * BT

        @pl.when(i + 3 < T)
        def _prefetch():
            start_copy(i + 3, jax.lax.rem(i + 3, 4))

        def gather_row(r, c):
            t = tok_sm[base + r]
            xs_ref[pl.ds(r, 1), :] = x_ref[pl.ds(t, 1), :]
            return c
        jax.lax.fori_loop(0, cnt, gather_row, 0)

        wait_copy(i, slot)
        e = te_ref[0, i]
        xs = xs_ref[...]
        gu = jnp.dot(xs, wgu_buf[slot], preferred_element_type=jnp.float32)
        gu = gu + bgu_ref[e, :][None, :]
        gate = jnp.minimum(gu[:, :I], LIMIT)
        up = jnp.clip(gu[:, I:], -LIMIT, LIMIT)
        glu = gate * jax.nn.sigmoid(gate * ALPHA)
        h = (up + 1.0) * glu
        y = jnp.dot(h, wdn_buf[slot], preferred_element_type=jnp.float32)
        y = y + bdn_ref[e, :][None, :]
        ys_ref[...] = y

        def scatter_row(r, c):
            t = tok_sm[base + r]
            w = ws_sm[base + r]
            out_ref[pl.ds(t, 1), :] += ys_ref[pl.ds(r, 1), :] * w
            return c
        jax.lax.fori_loop(0, cnt, scatter_row, 0)
        return carry

    jax.lax.fori_loop(0, T, tile_body, 0)


@functools.partial(jax.jit, static_argnames=())
def kernel(hidden_states, router_indices, routing_weights,
           W_gate_up, b_gate_up, W_down, b_down):
    x = hidden_states[0]
    ex2d = router_indices.astype(jnp.int32).reshape(NR, 128)

    dest, te, tc, tot = pl.pallas_call(
        _meta_body,
        out_shape=[
            jax.ShapeDtypeStruct((NR, 128), jnp.int32),
            jax.ShapeDtypeStruct((1, G), jnp.int32),
            jax.ShapeDtypeStruct((1, G), jnp.int32),
            jax.ShapeDtypeStruct((1, 1), jnp.int32),
        ],
    )(ex2d)

    grid_spec = pltpu.PrefetchScalarGridSpec(
        num_scalar_prefetch=5,
        grid=(1,),
        in_specs=[
            pl.BlockSpec((S, H), lambda g, *s: (0, 0)),
            pl.BlockSpec((E, 2 * I), lambda g, *s: (0, 0)),
            pl.BlockSpec((E, H), lambda g, *s: (0, 0)),
            pl.BlockSpec(memory_space=pl.ANY),
            pl.BlockSpec(memory_space=pl.ANY),
        ],
        out_specs=pl.BlockSpec((S, H), lambda g, *s: (0, 0)),
        scratch_shapes=[
            pltpu.VMEM((4, H, 2 * I), jnp.float32),
            pltpu.VMEM((4, I, H), jnp.float32),
            pltpu.VMEM((BT, H), jnp.float32),
            pltpu.VMEM((BT, H), jnp.float32),
            pltpu.SMEM((G * BT,), jnp.int32),
            pltpu.SMEM((G * BT,), jnp.float32),
            pltpu.SemaphoreType.DMA((4, 6)),
        ],
    )

    out = pl.pallas_call(
        _moe_body,
        grid_spec=grid_spec,
        out_shape=jax.ShapeDtypeStruct((S, H), jnp.float32),
        compiler_params=pltpu.CompilerParams(
            dimension_semantics=("arbitrary",),
        ),
    )(te, tc, tot, dest, routing_weights.reshape(NR, 128),
      x, b_gate_up, b_down, W_gate_up, W_down)

    return out.reshape(1, S, H)


# speedup vs baseline: 1.7699x; 1.0151x over previous
import functools

import jax
import jax.numpy as jnp
from jax.experimental import pallas as pl
from jax.experimental.pallas import tpu as pltpu

E = 64
TOPK = 2
H = 768
I = 768
S = 2048
N = S * TOPK
LIMIT = 7.0
ALPHA = 1.702

BT = 128
G = N // BT + E
NR = N // 128


def _meta_body(ex_ref, dest_ref, te_ref, tc_ref, tot_ref):
    ex = ex_ref[...]
    dest = jnp.zeros((NR, 128), jnp.int32)
    te = jnp.zeros((1, G), jnp.int32)
    tc = jnp.zeros((1, G), jnp.int32)
    lanes_g = jax.lax.broadcasted_iota(jnp.int32, (1, G), 1)
    ft = jnp.zeros((1, 1), jnp.int32)

    for e in range(E):
        m = (ex == e).astype(jnp.int32)
        c = m
        for k in (1, 2, 4, 8, 16, 32, 64):
            c = c + jnp.concatenate(
                [jnp.zeros((NR, k), jnp.int32), c[:, :128 - k]], axis=1)
        rs = c[:, 127:128]
        rc = rs
        kk = 1
        while kk < NR:
            rc = rc + jnp.concatenate(
                [jnp.zeros((kk, 1), jnp.int32), rc[:NR - kk]], axis=0)
            kk *= 2
        rank = c - m + (rc - rs)
        cnt = rc[NR - 1:NR, 0:1]
        nt = (cnt + BT - 1) >> 7
        dest = dest + m * (ft * BT + rank)
        in_e = (lanes_g >= ft) & (lanes_g < ft + nt)
        te = jnp.where(in_e, e, te)
        tc = jnp.where(in_e,
                       jnp.clip(cnt - (lanes_g - ft) * BT, 0, BT), tc)
        ft = ft + nt

    dest_ref[...] = dest
    te_ref[...] = te
    tc_ref[...] = tc
    tot_ref[...] = ft


def _moe_body(te_ref, tc_ref, tot_ref, dest_ref, rw_ref,
              x_ref, bgu_ref, bdn_ref, wgu_hbm, wdn_hbm,
              out_ref,
              wgu_buf, wdn_buf, xs_ref, ys_ref,
              tok_sm, ws_sm, dsem):
    T = tot_ref[0, 0]
    out_ref[...] = jnp.zeros_like(out_ref)

    CGU, CDN = 8, 4
    RGU, RDN = H // CGU, I // CDN

    def _copies(i, slot):
        e = te_ref[0, i]
        cps = []
        for c in range(CGU):
            cps.append(pltpu.make_async_copy(
                wgu_hbm.at[e, pl.ds(c * RGU, RGU)],
                wgu_buf.at[slot, pl.ds(c * RGU, RGU)],
                dsem.at[slot, c]))
        for c in range(CDN):
            cps.append(pltpu.make_async_copy(
                wdn_hbm.at[e, pl.ds(c * RDN, RDN)],
                wdn_buf.at[slot, pl.ds(c * RDN, RDN)],
                dsem.at[slot, CGU + c]))
        return cps

    def start_copy(i, slot):
        for cp in _copies(i, slot):
            cp.start()

    def wait_copy(i, slot):
        for cp in _copies(i, slot):
            cp.wait()

    start_copy(0, 0)
    for j in (1, 2, 3, 4):
        @pl.when(j < T)
        def _pro():
            start_copy(j, j)

    def invert(p, c):
        hi = jax.lax.shift_right_logical(p, 7)
        lo = jax.lax.bitwise_and(p, 127)
        s = dest_ref[hi, lo]
        tok_sm[s] = jax.lax.shift_right_logical(p, 1)
        ws_sm[s] = rw_ref[hi, lo]
        return c
    jax.lax.fori_loop(0, N, invert, 0, unroll=4)

    def tile_body(i, carry):
        slot = jax.lax.rem(i, 6)
        cnt = tc_ref[0, i]
        base = i * BT

        @pl.when(i + 5 < T)
        def _prefetch():
            start_copy(i + 5, jax.lax.rem(i + 5, 6))

        def gather_row(r, c):
            t = tok_sm[base + r]
            xs_ref[pl.ds(r, 1), :] = x_ref[pl.ds(t, 1), :]
            return c
        jax.lax.fori_loop(0, cnt, gather_row, 0)

        wait_copy(i, slot)
        e = te_ref[0, i]
        xs = xs_ref[...]
        gu = jnp.dot(xs, wgu_buf[slot], preferred_element_type=jnp.float32)
        gu = gu + bgu_ref[e, :][None, :]
        gate = jnp.minimum(gu[:, :I], LIMIT)
        up = jnp.clip(gu[:, I:], -LIMIT, LIMIT)
        glu = gate * jax.nn.sigmoid(gate * ALPHA)
        h = (up + 1.0) * glu
        y = jnp.dot(h, wdn_buf[slot], preferred_element_type=jnp.float32)
        y = y + bdn_ref[e, :][None, :]
        ys_ref[...] = y

        def scatter_row(r, c):
            t = tok_sm[base + r]
            w = ws_sm[base + r]
            out_ref[pl.ds(t, 1), :] += ys_ref[pl.ds(r, 1), :] * w
            return c
        jax.lax.fori_loop(0, cnt, scatter_row, 0)
        return carry

    jax.lax.fori_loop(0, T, tile_body, 0)


@functools.partial(jax.jit, static_argnames=())
def kernel(hidden_states, router_indices, routing_weights,
           W_gate_up, b_gate_up, W_down, b_down):
    x = hidden_states[0]
    ex2d = router_indices.astype(jnp.int32).reshape(NR, 128)

    dest, te, tc, tot = pl.pallas_call(
        _meta_body,
        out_shape=[
            jax.ShapeDtypeStruct((NR, 128), jnp.int32),
            jax.ShapeDtypeStruct((1, G), jnp.int32),
            jax.ShapeDtypeStruct((1, G), jnp.int32),
            jax.ShapeDtypeStruct((1, 1), jnp.int32),
        ],
    )(ex2d)

    grid_spec = pltpu.PrefetchScalarGridSpec(
        num_scalar_prefetch=5,
        grid=(1,),
        in_specs=[
            pl.BlockSpec((S, H), lambda g, *s: (0, 0)),
            pl.BlockSpec((E, 2 * I), lambda g, *s: (0, 0)),
            pl.BlockSpec((E, H), lambda g, *s: (0, 0)),
            pl.BlockSpec(memory_space=pl.ANY),
            pl.BlockSpec(memory_space=pl.ANY),
        ],
        out_specs=pl.BlockSpec((S, H), lambda g, *s: (0, 0)),
        scratch_shapes=[
            pltpu.VMEM((6, H, 2 * I), jnp.float32),
            pltpu.VMEM((6, I, H), jnp.float32),
            pltpu.VMEM((BT, H), jnp.float32),
            pltpu.VMEM((BT, H), jnp.float32),
            pltpu.SMEM((G * BT,), jnp.int32),
            pltpu.SMEM((G * BT,), jnp.float32),
            pltpu.SemaphoreType.DMA((6, 12)),
        ],
    )

    out = pl.pallas_call(
        _moe_body,
        grid_spec=grid_spec,
        out_shape=jax.ShapeDtypeStruct((S, H), jnp.float32),
        compiler_params=pltpu.CompilerParams(
            dimension_semantics=("arbitrary",),
        ),
    )(te, tc, tot, dest, routing_weights.reshape(NR, 128),
      x, b_gate_up, b_down, W_gate_up, W_down)

    return out.reshape(1, S, H)
